# baseline (speedup 1.0000x reference)
import functools

import jax
import jax.numpy as jnp
from jax import lax
from jax.experimental import pallas as pl
from jax.experimental.pallas import tpu as pltpu
from jax.experimental.pallas import tpu_sc as plsc

N = 10000
E = 320000
D = 128
L = 4

NC, NS, LANES = 2, 16, 16
NW = NC * NS
NP = 313
NPAD = NW * NP
DUMP = NP
SB = 14
DMASK = (1 << SB) - 1

EC = 16000
F = 16384
S = F + EC + 784
SH = S - F
CAP = E + 2 * F
GCH = 256
MAXSORT = F + 16256
HB = 352

_sc_params = pltpu.CompilerParams(needs_layout_passes=False)
_mesh = plsc.VectorSubcoreMesh(core_axis_name="c", subcore_axis_name="s")


@functools.partial(
    pl.kernel,
    mesh=_mesh,
    compiler_params=_sc_params,
    out_type=(
        jax.ShapeDtypeStruct((NW * CAP,), jnp.int32),
        jax.ShapeDtypeStruct((NW * 128,), jnp.int32),
    ),
    scratch_types=[
        pltpu.VMEM((EC + LANES,), jnp.int32),
        pltpu.VMEM((S,), jnp.int32),
        pltpu.VMEM((HB,), jnp.int32),
        pltpu.VMEM((HB,), jnp.int32),
        pltpu.VMEM((HB,), jnp.int32),
        pltpu.VMEM((HB,), jnp.int32),
        pltpu.VMEM((HB,), jnp.int32),
        pltpu.VMEM((HB,), jnp.int32),
        pltpu.VMEM((HB,), jnp.int32),
        pltpu.VMEM((HB,), jnp.int32),
    ],
)
def _bin_edges(pk_hbm, bpk, bcnt, pk_c, st_pk, h0, h1, h2, h3, o0, o1, o2, o3):
    wid = lax.axis_index("s") * NC + lax.axis_index("c")
    lo = wid * NP
    iota = jnp.arange(LANES, dtype=jnp.int32)
    lane0 = iota == 0
    z16 = jnp.zeros((LANES,), jnp.int32)

    def outer(g, carry):
        cnt_st, flushed = carry
        eoff = pl.multiple_of(g * EC, 128)
        pltpu.sync_copy(pk_hbm.at[pl.ds(eoff, EC)], pk_c.at[pl.ds(0, EC)])

        def inner(i, cnt):
            pk = pk_c[pl.ds(i * LANES, LANES)]
            dl = (pk & DMASK) - lo
            m = (dl >= 0) & (dl < NP)
            plsc.store_compressed(st_pk.at[pl.ds(cnt, LANES)], pk - lo, mask=m)
            return cnt + plsc.all_reduce_population_count(m)[0]

        cnt_st = lax.fori_loop(0, EC // LANES, inner, cnt_st)

        do = cnt_st >= F

        @pl.when(do)
        def _flush():
            off = pl.multiple_of(wid * CAP + flushed, 128)
            pltpu.sync_copy(st_pk.at[pl.ds(0, F)], bpk.at[pl.ds(off, F)])

            def shift(i, c):
                st_pk[pl.ds(i * LANES, LANES)] = st_pk[pl.ds(F + i * LANES, LANES)]
                return c

            lax.fori_loop(0, SH // LANES, shift, 0)

        cnt_st = jnp.where(do, cnt_st - F, cnt_st)
        flushed = jnp.where(do, flushed + F, flushed)
        return cnt_st, flushed

    cnt_st, flushed = lax.fori_loop(0, E // EC, outer, (0, 0))

    pad_n = lax.rem(2 * GCH - lax.rem(cnt_st, 2 * GCH), 2 * GCH)
    pad_pk = z16 + (wid * (1 << SB) + DUMP)
    for j in range(2 * GCH // LANES):
        @pl.when(j * LANES < pad_n)
        def _pad():
            st_pk[pl.ds(cnt_st + j * LANES, LANES)] = pad_pk

    off = pl.multiple_of(wid * CAP + flushed, 128)
    pltpu.sync_copy(st_pk.at[pl.ds(0, F)], bpk.at[pl.ds(off, F)])

    total = flushed + cnt_st + pad_n
    sortable = total <= MAXSORT

    @pl.when(sortable)
    def _sort():
        hs = (h0, h1, h2, h3)
        os_ = (o0, o1, o2, o3)
        for hk in hs:
            for k in range(HB // LANES):
                hk[pl.ds(k * LANES, LANES)] = z16

        nct = lax.div(total + (EC - 1), EC)

        def hist_chunk(t, c):
            coff = pl.multiple_of(wid * CAP + t * EC, 128)
            pltpu.sync_copy(bpk.at[pl.ds(coff, EC)], pk_c.at[pl.ds(0, EC)])
            nb = jnp.minimum(EC, total - t * EC)

            def hist_edge(i, cc):
                for k in range(4):
                    d = pk_c[pl.ds(i * 4 + k, LANES)][0] & DMASK
                    hcnt = hs[k][pl.ds(d, LANES)][0]
                    plsc.store_scatter(hs[k], [z16 + d], z16 + (hcnt + 1), mask=lane0)
                return cc

            lax.fori_loop(0, lax.div(nb, 4), hist_edge, 0)
            return c

        lax.fori_loop(0, nct, hist_chunk, 0)

        running = jnp.int32(0)
        for k in range(HB // LANES):
            sl = pl.ds(k * LANES, LANES)
            v0, v1, v2, v3 = h0[sl], h1[sl], h2[sl], h3[sl]
            hv = v0 + v1 + v2 + v3
            cs = plsc.cumsum(hv)
            base = cs - hv + running
            o0[sl] = base
            o1[sl] = base + v0
            o2[sl] = base + v0 + v1
            o3[sl] = base + v0 + v1 + v2
            running = running + cs[LANES - 1]

        def perm_chunk(t, c):
            coff = pl.multiple_of(wid * CAP + t * EC, 128)
            pltpu.sync_copy(bpk.at[pl.ds(coff, EC)], pk_c.at[pl.ds(0, EC)])
            nb = jnp.minimum(EC, total - t * EC)

            def perm_edge(i, cc):
                for k in range(4):
                    p = pk_c[pl.ds(i * 4 + k, LANES)][0]
                    d = p & DMASK
                    o = os_[k][pl.ds(d, LANES)][0]
                    plsc.store_scatter(os_[k], [z16 + d], z16 + (o + 1), mask=lane0)
                    plsc.store_scatter(st_pk, [z16 + o], z16 + p, mask=lane0)
                return cc

            lax.fori_loop(0, lax.div(nb, 4), perm_edge, 0)
            return c

        lax.fori_loop(0, nct, perm_chunk, 0)

        base = pl.multiple_of(wid * CAP, 128)
        pltpu.sync_copy(st_pk.at[pl.ds(0, F)], bpk.at[pl.ds(base, F)])
        base2 = pl.multiple_of(wid * CAP + F, 128)
        pltpu.sync_copy(st_pk.at[pl.ds(F, MAXSORT - F)], bpk.at[pl.ds(base2, MAXSORT - F)])

    flag = jnp.where(sortable, 1, 0)
    bcnt_v = jnp.where(iota == 0, z16 + total, jnp.where(iota == 1, z16 + flag, z16))
    pk_c[pl.ds(0, LANES)] = bcnt_v
    pltpu.sync_copy(pk_c.at[pl.ds(0, LANES)], bcnt.at[pl.ds(pl.multiple_of(wid * 128, 128), LANES)])


_NEG = float("-inf")


@functools.partial(
    pl.kernel,
    mesh=_mesh,
    compiler_params=_sc_params,
    out_type=jax.ShapeDtypeStruct((NW, NP, D), jnp.float32),
    scratch_types=[
        pltpu.VMEM((NW + LANES,), jnp.int32),
        pltpu.VMEM((GCH + LANES,), jnp.int32),
        pltpu.VMEM((GCH + LANES,), jnp.int32),
        pltpu.VMEM((GCH + LANES,), jnp.int32),
        pltpu.VMEM((GCH + LANES,), jnp.int32),
        pltpu.VMEM((GCH + LANES,), jnp.int32),
        pltpu.VMEM((GCH + LANES,), jnp.int32),
        pltpu.VMEM((GCH + LANES,), jnp.int32),
        pltpu.VMEM((GCH + LANES,), jnp.int32),
        pltpu.VMEM((GCH + LANES,), jnp.int32),
        pltpu.VMEM((GCH + LANES,), jnp.int32),
        pltpu.VMEM((GCH,), jnp.int32),
        pltpu.VMEM((GCH,), jnp.int32),
        pltpu.VMEM((GCH, D), jnp.float32),
        pltpu.VMEM((GCH, D), jnp.float32),
        pltpu.VMEM((NP + 1, D), jnp.float32),
        pltpu.SemaphoreType.DMA,
        pltpu.SemaphoreType.DMA,
        pltpu.SemaphoreType.DMA,
        pltpu.SemaphoreType.DMA,
    ],
)
def _seg_max(u_hbm, bpk, bcnt, neg_hbm, agg, cnt_v, pk0, pk1, dl0, dl1,
             dp0, dp1, sm0, sm1, ls0, ls1, sidx0, sidx1, rows0, rows1, acc,
             p0, p1, r0, r1):
    wid = lax.axis_index("s") * NC + lax.axis_index("c")

    pltpu.sync_copy(bcnt.at[pl.ds(pl.multiple_of(wid * 128, 128), LANES)], cnt_v.at[pl.ds(0, LANES)])
    hdr = cnt_v[pl.ds(0, LANES)]
    cnt = hdr[0]
    flag = hdr[1]
    pltpu.sync_copy(neg_hbm, acc)
    nch = lax.div(cnt, GCH)

    pk = (pk0, pk1)
    dloc = (dl0, dl1)
    dpv = (dp0, dp1)
    sames = (sm0, sm1)
    lasts = (ls0, ls1)
    sidx = (sidx0, sidx1)
    rows = (rows0, rows1)
    psem = (p0, p1)
    rsem = (r0, r1)

    def idx_off(g):
        return pl.multiple_of(wid * CAP + g * GCH, 128)

    iota16 = jnp.arange(LANES, dtype=jnp.int32)
    z16i = jnp.zeros((LANES,), jnp.int32)

    def unpack(b, prev_last):
        for k in range(GCH // LANES):
            sl = pl.ds(k * LANES, LANES)
            w = pk[b][sl]
            sidx[b][sl] = w >> SB
            d = w & DMASK
            dloc[b][sl] = d
            dpv[b][pl.ds(k * LANES + 1, LANES)] = d
        plsc.store_scatter(dpv[b], [z16i], z16i + prev_last, mask=iota16 == 0)
        for k in range(GCH // LANES):
            sl = pl.ds(k * LANES, LANES)
            sames[b][sl] = jnp.where(dloc[b][sl] == dpv[b][sl], 1, 0)
        for k in range(GCH // LANES):
            lasts[b][pl.ds(k * LANES, LANES)] = 1 - sames[b][pl.ds(k * LANES + 1, LANES)]
        plsc.store_scatter(lasts[b], [z16i + (GCH - 1)], z16i + 1, mask=iota16 == 0)

    @pl.when(flag == 1)
    def _sorted():
        @pl.when(nch >= 1)
        def _pro0():
            pltpu.async_copy(bpk.at[pl.ds(idx_off(0), GCH)], pk0.at[pl.ds(0, GCH)], p0).wait()
            unpack(0, jnp.int32(-1))
            pltpu.async_copy(u_hbm.at[sidx0], rows0, r0)

        @pl.when(nch >= 2)
        def _pro1():
            pltpu.async_copy(bpk.at[pl.ds(idx_off(1), GCH)], pk1.at[pl.ds(0, GCH)], p1)

        def pair(g2, carry):
            for b in (0, 1):
                g = g2 * 2 + b
                nbuf = 1 - b
                pltpu.make_async_copy(u_hbm.at[sidx[b]], rows[b], rsem[b]).wait()

                @pl.when(g + 1 < nch)
                def _launch_next_gather():
                    pltpu.make_async_copy(
                        bpk.at[pl.ds(idx_off(g + 1), GCH)], pk[nbuf].at[pl.ds(0, GCH)], psem[nbuf]
                    ).wait()
                    unpack(nbuf, dloc[b][pl.ds(GCH - 1, LANES)][0])
                    plsc.store_scatter(
                        lasts[b], [z16i + (GCH - 1)],
                        z16i + (1 - sames[nbuf][pl.ds(0, LANES)][0]),
                        mask=iota16 == 0,
                    )
                    pltpu.async_copy(u_hbm.at[sidx[nbuf]], rows[nbuf], rsem[nbuf])

                def body(e, ms):
                    same = sames[b][pl.ds(e, LANES)][0] != 0
                    new_ms = []
                    for r in range(D // LANES):
                        sl = pl.ds(r * LANES, LANES)
                        row_r = rows[b][e, sl]
                        new_ms.append(jnp.where(same, jnp.maximum(ms[r], row_r), row_r))
                    lst = lasts[b][pl.ds(e, LANES)][0] != 0

                    @pl.when(lst)
                    def _store():
                        dst = dloc[b][pl.ds(e, LANES)][0]
                        for r in range(D // LANES):
                            acc[dst, pl.ds(r * LANES, LANES)] = new_ms[r]

                    return tuple(new_ms)

                carry = lax.fori_loop(0, GCH, body, carry)

                @pl.when(g + 2 < nch)
                def _prefetch_pk():
                    pltpu.async_copy(
                        bpk.at[pl.ds(idx_off(g + 2), GCH)], pk[b].at[pl.ds(0, GCH)], psem[b]
                    )

            return carry

        init = tuple(
            jnp.full((LANES,), _NEG, jnp.float32) for _ in range(D // LANES)
        )
        lax.fori_loop(0, lax.div(nch, 2), pair, init)

    @pl.when(flag == 0)
    def _rmw():
        def chunk(g, carry):
            pltpu.sync_copy(bpk.at[pl.ds(idx_off(g), GCH)], pk0.at[pl.ds(0, GCH)])
            unpack(0, jnp.int32(-1))
            pltpu.async_copy(u_hbm.at[sidx0], rows0, r0).wait()

            def body(e, c):
                dst = dl0[pl.ds(e, LANES)][0]
                for r in range(D // LANES):
                    sl = pl.ds(r * LANES, LANES)
                    acc[dst, sl] = jnp.maximum(acc[dst, sl], rows0[e, sl])
                return c

            lax.fori_loop(0, GCH, body, 0)
            return carry

        lax.fori_loop(0, nch, chunk, 0)

    pltpu.sync_copy(acc.at[pl.ds(0, NP)], agg.at[wid])


def _mm_first(x_ref, w1_ref, w2_ref, b_ref, u_ref, v_ref):
    x = x_ref[...]
    u_ref[...] = jnp.dot(x, w1_ref[...], preferred_element_type=jnp.float32)
    v_ref[...] = (
        jnp.dot(x, w2_ref[...], preferred_element_type=jnp.float32) + b_ref[0:1]
    )


def _mm_fused(agg_ref, vin_ref, w1_ref, w2_ref, b_ref, u_ref, v_ref):
    x = jnp.maximum(agg_ref[...] + vin_ref[...], 0.0)
    u_ref[...] = jnp.dot(x, w1_ref[...], preferred_element_type=jnp.float32)
    v_ref[...] = (
        jnp.dot(x, w2_ref[...], preferred_element_type=jnp.float32) + b_ref[0:1]
    )


def _epilogue(agg_ref, vin_ref, y_ref):
    y_ref[...] = jnp.maximum(agg_ref[...] + vin_ref[...], 0.0)


_RB = NPAD // 4


def _row_spec():
    return pl.BlockSpec((_RB, D), lambda i: (i, 0))


def _full_spec(shape):
    return pl.BlockSpec(shape, lambda i: tuple(0 for _ in shape))


_mm_first_call = pl.pallas_call(
    _mm_first,
    grid=(4,),
    in_specs=[_row_spec(), _full_spec((D, D)), _full_spec((D, D)), _full_spec((8, D))],
    out_specs=[_row_spec(), _row_spec()],
    out_shape=(
        jax.ShapeDtypeStruct((NPAD, D), jnp.float32),
        jax.ShapeDtypeStruct((NPAD, D), jnp.float32),
    ),
)

_mm_fused_call = pl.pallas_call(
    _mm_fused,
    grid=(4,),
    in_specs=[
        _row_spec(),
        _row_spec(),
        _full_spec((D, D)),
        _full_spec((D, D)),
        _full_spec((8, D)),
    ],
    out_specs=[_row_spec(), _row_spec()],
    out_shape=(
        jax.ShapeDtypeStruct((NPAD, D), jnp.float32),
        jax.ShapeDtypeStruct((NPAD, D), jnp.float32),
    ),
)

_epilogue_call = pl.pallas_call(
    _epilogue,
    grid=(4,),
    in_specs=[_row_spec(), _row_spec()],
    out_specs=_row_spec(),
    out_shape=jax.ShapeDtypeStruct((NPAD, D), jnp.float32),
)


def kernel(feats, graph, theta_w, theta_b, phi_w, phi_b):
    src = graph[0].astype(jnp.int32)
    dst = graph[1].astype(jnp.int32)
    pk_in = (src << SB) | dst

    bpk, bcnt = _bin_edges(pk_in)

    w1 = jnp.transpose(theta_w, (0, 2, 1))
    w2 = jnp.transpose(phi_w - theta_w, (0, 2, 1))
    b = jnp.broadcast_to((theta_b + phi_b).reshape(L, 1, D), (L, 8, D))

    xpad = jnp.concatenate(
        [feats, jnp.zeros((NPAD - N, D), jnp.float32)], axis=0
    )
    neg = jnp.full((NP + 1, D), -jnp.inf, jnp.float32)

    u, v = _mm_first_call(xpad, w1[0], w2[0], b[0])
    for l in range(1, L):
        agg = _seg_max(u, bpk, bcnt, neg)
        agg = agg.reshape(NPAD, D)
        u, v = _mm_fused_call(agg, v, w1[l], w2[l], b[l])
    agg = _seg_max(u, bpk, bcnt, neg)
    y = _epilogue_call(agg.reshape(NPAD, D), v)
    return y[:N]


# speedup vs baseline: 1.3213x; 1.3213x over previous
import functools

import jax
import jax.numpy as jnp
from jax import lax
from jax.experimental import pallas as pl
from jax.experimental.pallas import tpu as pltpu
from jax.experimental.pallas import tpu_sc as plsc

N = 10000
E = 320000
D = 128
L = 4

NC, NS, LANES = 2, 16, 16
NW = NC * NS
NP = 313
NPAD = NW * NP
DUMP = NP

EC = 16000
F = 16384
S = F + EC + 784
SH = S - F
CAP = E + 2 * F
GCH = 256
MAXSORT = F + 16256
HB = 352

_sc_params = pltpu.CompilerParams(needs_layout_passes=False)
_mesh = plsc.VectorSubcoreMesh(core_axis_name="c", subcore_axis_name="s")


@functools.partial(
    pl.kernel,
    mesh=_mesh,
    compiler_params=_sc_params,
    out_type=(
        jax.ShapeDtypeStruct((NW * CAP,), jnp.int32),
        jax.ShapeDtypeStruct((NW * CAP,), jnp.int32),
        jax.ShapeDtypeStruct((NW * 128,), jnp.int32),
    ),
    scratch_types=[
        pltpu.VMEM((EC + LANES,), jnp.int32),
        pltpu.VMEM((EC + LANES,), jnp.int32),
        pltpu.VMEM((S,), jnp.int32),
        pltpu.VMEM((S,), jnp.int32),
        pltpu.VMEM((HB,), jnp.int32),
        pltpu.VMEM((HB,), jnp.int32),
        pltpu.VMEM((HB,), jnp.int32),
        pltpu.VMEM((HB,), jnp.int32),
        pltpu.VMEM((HB,), jnp.int32),
        pltpu.VMEM((HB,), jnp.int32),
        pltpu.VMEM((HB,), jnp.int32),
        pltpu.VMEM((HB,), jnp.int32),
    ],
)
def _bin_edges(src_hbm, dst_hbm, bsrc, bdst, bcnt, src_c, dst_c, st_src, st_dst,
               h0, h1, h2, h3, o0, o1, o2, o3):
    wid = lax.axis_index("s") * NC + lax.axis_index("c")
    lo = wid * NP
    iota = jnp.arange(LANES, dtype=jnp.int32)
    lane0 = iota == 0
    z16 = jnp.zeros((LANES,), jnp.int32)

    def outer(g, carry):
        cnt_st, flushed = carry
        eoff = pl.multiple_of(g * EC, 128)
        pltpu.sync_copy(src_hbm.at[pl.ds(eoff, EC)], src_c.at[pl.ds(0, EC)])
        pltpu.sync_copy(dst_hbm.at[pl.ds(eoff, EC)], dst_c.at[pl.ds(0, EC)])

        def inner(i, cnt):
            sl = pl.ds(i * LANES, LANES)
            s = src_c[sl]
            dl = dst_c[sl] - lo
            m = (dl >= 0) & (dl < NP)
            plsc.store_compressed(st_src.at[pl.ds(cnt, LANES)], s, mask=m)
            plsc.store_compressed(st_dst.at[pl.ds(cnt, LANES)], dl, mask=m)
            return cnt + plsc.all_reduce_population_count(m)[0]

        cnt_st = lax.fori_loop(0, EC // LANES, inner, cnt_st)

        do = cnt_st >= F

        @pl.when(do)
        def _flush():
            off = pl.multiple_of(wid * CAP + flushed, 128)
            pltpu.sync_copy(st_src.at[pl.ds(0, F)], bsrc.at[pl.ds(off, F)])
            pltpu.sync_copy(st_dst.at[pl.ds(0, F)], bdst.at[pl.ds(off, F)])

            def shift(i, c):
                sl_lo = pl.ds(i * LANES, LANES)
                sl_hi = pl.ds(F + i * LANES, LANES)
                st_src[sl_lo] = st_src[sl_hi]
                st_dst[sl_lo] = st_dst[sl_hi]
                return c

            lax.fori_loop(0, SH // LANES, shift, 0)

        cnt_st = jnp.where(do, cnt_st - F, cnt_st)
        flushed = jnp.where(do, flushed + F, flushed)
        return cnt_st, flushed

    cnt_st, flushed = lax.fori_loop(0, E // EC, outer, (0, 0))

    pad_n = lax.rem(2 * GCH - lax.rem(cnt_st, 2 * GCH), 2 * GCH)
    pad_src = z16 + wid
    pad_dst = z16 + DUMP
    for j in range(2 * GCH // LANES):
        @pl.when(j * LANES < pad_n)
        def _pad():
            st_src[pl.ds(cnt_st + j * LANES, LANES)] = pad_src
            st_dst[pl.ds(cnt_st + j * LANES, LANES)] = pad_dst

    off = pl.multiple_of(wid * CAP + flushed, 128)
    pltpu.sync_copy(st_src.at[pl.ds(0, F)], bsrc.at[pl.ds(off, F)])
    pltpu.sync_copy(st_dst.at[pl.ds(0, F)], bdst.at[pl.ds(off, F)])

    total = flushed + cnt_st + pad_n
    sortable = total <= MAXSORT

    @pl.when(sortable)
    def _sort():
        hs = (h0, h1, h2, h3)
        os_ = (o0, o1, o2, o3)
        for hk in hs:
            for k in range(HB // LANES):
                hk[pl.ds(k * LANES, LANES)] = z16

        nct = lax.div(total + (EC - 1), EC)

        def hist_chunk(t, c):
            coff = pl.multiple_of(wid * CAP + t * EC, 128)
            pltpu.sync_copy(bdst.at[pl.ds(coff, EC)], dst_c.at[pl.ds(0, EC)])
            nb = jnp.minimum(EC, total - t * EC)

            def hist_edge(i, cc):
                for k in range(4):
                    d = dst_c[pl.ds(i * 4 + k, LANES)][0]
                    hcnt = hs[k][pl.ds(d, LANES)][0]
                    plsc.store_scatter(hs[k], [z16 + d], z16 + (hcnt + 1), mask=lane0)
                return cc

            lax.fori_loop(0, lax.div(nb, 4), hist_edge, 0)
            return c

        lax.fori_loop(0, nct, hist_chunk, 0)

        running = jnp.int32(0)
        for k in range(HB // LANES):
            sl = pl.ds(k * LANES, LANES)
            v0, v1, v2, v3 = h0[sl], h1[sl], h2[sl], h3[sl]
            hv = v0 + v1 + v2 + v3
            cs = plsc.cumsum(hv)
            base = cs - hv + running
            o0[sl] = base
            o1[sl] = base + v0
            o2[sl] = base + v0 + v1
            o3[sl] = base + v0 + v1 + v2
            running = running + cs[LANES - 1]

        def perm_chunk(t, c):
            coff = pl.multiple_of(wid * CAP + t * EC, 128)
            pltpu.sync_copy(bsrc.at[pl.ds(coff, EC)], src_c.at[pl.ds(0, EC)])
            pltpu.sync_copy(bdst.at[pl.ds(coff, EC)], dst_c.at[pl.ds(0, EC)])
            nb = jnp.minimum(EC, total - t * EC)

            def perm_edge(i, cc):
                for k in range(4):
                    s = src_c[pl.ds(i * 4 + k, LANES)][0]
                    d = dst_c[pl.ds(i * 4 + k, LANES)][0]
                    o = os_[k][pl.ds(d, LANES)][0]
                    plsc.store_scatter(os_[k], [z16 + d], z16 + (o + 1), mask=lane0)
                    plsc.store_scatter(st_src, [z16 + o], z16 + s, mask=lane0)
                    plsc.store_scatter(st_dst, [z16 + o], z16 + d, mask=lane0)
                return cc

            lax.fori_loop(0, lax.div(nb, 4), perm_edge, 0)
            return c

        lax.fori_loop(0, nct, perm_chunk, 0)

        base = pl.multiple_of(wid * CAP, 128)
        pltpu.sync_copy(st_src.at[pl.ds(0, F)], bsrc.at[pl.ds(base, F)])
        pltpu.sync_copy(st_dst.at[pl.ds(0, F)], bdst.at[pl.ds(base, F)])
        base2 = pl.multiple_of(wid * CAP + F, 128)
        pltpu.sync_copy(st_src.at[pl.ds(F, MAXSORT - F)], bsrc.at[pl.ds(base2, MAXSORT - F)])
        pltpu.sync_copy(st_dst.at[pl.ds(F, MAXSORT - F)], bdst.at[pl.ds(base2, MAXSORT - F)])

    flag = jnp.where(sortable, 1, 0)
    bcnt_v = jnp.where(iota == 0, z16 + total, jnp.where(iota == 1, z16 + flag, z16))
    src_c[pl.ds(0, LANES)] = bcnt_v
    pltpu.sync_copy(src_c.at[pl.ds(0, LANES)], bcnt.at[pl.ds(pl.multiple_of(wid * 128, 128), LANES)])


_NEG = float("-inf")


@functools.partial(
    pl.kernel,
    mesh=_mesh,
    compiler_params=_sc_params,
    out_type=jax.ShapeDtypeStruct((NW, NP, D), jnp.float32),
    scratch_types=[
        pltpu.VMEM((NW + LANES,), jnp.int32),
        pltpu.VMEM((GCH,), jnp.int32),
        pltpu.VMEM((GCH,), jnp.int32),
        pltpu.VMEM((GCH + LANES,), jnp.int32),
        pltpu.VMEM((GCH + LANES,), jnp.int32),
        pltpu.VMEM((GCH, D), jnp.float32),
        pltpu.VMEM((GCH, D), jnp.float32),
        pltpu.VMEM((NP + 1, D), jnp.float32),
        pltpu.SemaphoreType.DMA,
        pltpu.SemaphoreType.DMA,
        pltpu.SemaphoreType.DMA,
        pltpu.SemaphoreType.DMA,
        pltpu.SemaphoreType.DMA,
        pltpu.SemaphoreType.DMA,
    ],
)
def _seg_max(u_hbm, bsrc, bdst, bcnt, neg_hbm, agg, cnt_v, sidx0, sidx1,
             dloc0, dloc1, rows0, rows1, acc, s0, s1, d0, d1, r0, r1):
    wid = lax.axis_index("s") * NC + lax.axis_index("c")

    pltpu.sync_copy(bcnt.at[pl.ds(pl.multiple_of(wid * 128, 128), LANES)], cnt_v.at[pl.ds(0, LANES)])
    hdr = cnt_v[pl.ds(0, LANES)]
    cnt = hdr[0]
    flag = hdr[1]
    pltpu.sync_copy(neg_hbm, acc)
    nch = lax.div(cnt, GCH)

    sidx = (sidx0, sidx1)
    dloc = (dloc0, dloc1)
    rows = (rows0, rows1)
    ssem = (s0, s1)
    dsem = (d0, d1)
    rsem = (r0, r1)

    def idx_off(g):
        return pl.multiple_of(wid * CAP + g * GCH, 128)

    @pl.when(flag == 1)
    def _sorted():
        @pl.when(nch >= 1)
        def _pro0():
            hs = pltpu.async_copy(bsrc.at[pl.ds(idx_off(0), GCH)], sidx0, s0)
            pltpu.async_copy(bdst.at[pl.ds(idx_off(0), GCH)], dloc0.at[pl.ds(0, GCH)], d0)
            hs.wait()
            pltpu.async_copy(u_hbm.at[sidx0], rows0, r0)

        @pl.when(nch >= 2)
        def _pro1():
            pltpu.async_copy(bsrc.at[pl.ds(idx_off(1), GCH)], sidx1, s1)
            pltpu.async_copy(bdst.at[pl.ds(idx_off(1), GCH)], dloc1.at[pl.ds(0, GCH)], d1)

        def pair(g2, carry):
            for b in (0, 1):
                g = g2 * 2 + b
                nbuf = 1 - b
                pltpu.make_async_copy(u_hbm.at[sidx[b]], rows[b], rsem[b]).wait()

                @pl.when(g + 1 < nch)
                def _launch_next_gather():
                    pltpu.make_async_copy(
                        bsrc.at[pl.ds(idx_off(g + 1), GCH)], sidx[nbuf], ssem[nbuf]
                    ).wait()
                    pltpu.async_copy(u_hbm.at[sidx[nbuf]], rows[nbuf], rsem[nbuf])

                @pl.when(g + 2 < nch)
                def _prefetch_sidx():
                    pltpu.async_copy(
                        bsrc.at[pl.ds(idx_off(g + 2), GCH)], sidx[b], ssem[b]
                    )

                pltpu.make_async_copy(
                    bdst.at[pl.ds(idx_off(g), GCH)], dloc[b].at[pl.ds(0, GCH)], dsem[b]
                ).wait()

                def body(e, car):
                    prev = car[0]
                    ms = car[1:]
                    dst = dloc[b][pl.ds(e, LANES)][0]
                    same = dst == prev
                    new_ms = []
                    for r in range(D // LANES):
                        sl = pl.ds(r * LANES, LANES)
                        row_r = rows[b][e, sl]
                        new_ms.append(jnp.where(same, jnp.maximum(ms[r], row_r), row_r))
                    for r in range(D // LANES):
                        acc[dst, pl.ds(r * LANES, LANES)] = new_ms[r]
                    return (dst, *new_ms)

                carry = lax.fori_loop(0, GCH, body, carry)

                @pl.when(g + 2 < nch)
                def _prefetch_dloc():
                    pltpu.async_copy(
                        bdst.at[pl.ds(idx_off(g + 2), GCH)], dloc[b].at[pl.ds(0, GCH)], dsem[b]
                    )

            return carry

        init = (jnp.int32(-1),) + tuple(
            jnp.full((LANES,), _NEG, jnp.float32) for _ in range(D // LANES)
        )
        lax.fori_loop(0, lax.div(nch, 2), pair, init)

    @pl.when(flag == 0)
    def _rmw():
        def chunk(g, carry):
            pltpu.sync_copy(bsrc.at[pl.ds(idx_off(g), GCH)], sidx0)
            pltpu.sync_copy(bdst.at[pl.ds(idx_off(g), GCH)], dloc0.at[pl.ds(0, GCH)])
            pltpu.async_copy(u_hbm.at[sidx0], rows0, r0).wait()

            def body(e, c):
                dst = dloc0[pl.ds(e, LANES)][0]
                for r in range(D // LANES):
                    sl = pl.ds(r * LANES, LANES)
                    acc[dst, sl] = jnp.maximum(acc[dst, sl], rows0[e, sl])
                return c

            lax.fori_loop(0, GCH, body, 0)
            return carry

        lax.fori_loop(0, nch, chunk, 0)

    pltpu.sync_copy(acc.at[pl.ds(0, NP)], agg.at[wid])


def _mm_first(x_ref, w1_ref, w2_ref, b_ref, u_ref, v_ref):
    x = x_ref[...]
    u_ref[...] = jnp.dot(x, w1_ref[...], preferred_element_type=jnp.float32)
    v_ref[...] = (
        jnp.dot(x, w2_ref[...], preferred_element_type=jnp.float32) + b_ref[0:1]
    )


def _mm_fused(agg_ref, vin_ref, w1_ref, w2_ref, b_ref, u_ref, v_ref):
    x = jnp.maximum(agg_ref[...] + vin_ref[...], 0.0)
    u_ref[...] = jnp.dot(x, w1_ref[...], preferred_element_type=jnp.float32)
    v_ref[...] = (
        jnp.dot(x, w2_ref[...], preferred_element_type=jnp.float32) + b_ref[0:1]
    )


def _epilogue(agg_ref, vin_ref, y_ref):
    y_ref[...] = jnp.maximum(agg_ref[...] + vin_ref[...], 0.0)


_RB = NPAD // 4


def _row_spec():
    return pl.BlockSpec((_RB, D), lambda i: (i, 0))


def _full_spec(shape):
    return pl.BlockSpec(shape, lambda i: tuple(0 for _ in shape))


_mm_first_call = pl.pallas_call(
    _mm_first,
    grid=(4,),
    in_specs=[_row_spec(), _full_spec((D, D)), _full_spec((D, D)), _full_spec((8, D))],
    out_specs=[_row_spec(), _row_spec()],
    out_shape=(
        jax.ShapeDtypeStruct((NPAD, D), jnp.float32),
        jax.ShapeDtypeStruct((NPAD, D), jnp.float32),
    ),
)

_mm_fused_call = pl.pallas_call(
    _mm_fused,
    grid=(4,),
    in_specs=[
        _row_spec(),
        _row_spec(),
        _full_spec((D, D)),
        _full_spec((D, D)),
        _full_spec((8, D)),
    ],
    out_specs=[_row_spec(), _row_spec()],
    out_shape=(
        jax.ShapeDtypeStruct((NPAD, D), jnp.float32),
        jax.ShapeDtypeStruct((NPAD, D), jnp.float32),
    ),
)

_epilogue_call = pl.pallas_call(
    _epilogue,
    grid=(4,),
    in_specs=[_row_spec(), _row_spec()],
    out_specs=_row_spec(),
    out_shape=jax.ShapeDtypeStruct((NPAD, D), jnp.float32),
)


def kernel(feats, graph, theta_w, theta_b, phi_w, phi_b):
    src = graph[0].astype(jnp.int32)
    dst = graph[1].astype(jnp.int32)

    bsrc, bdst, bcnt = _bin_edges(src, dst)

    w1 = jnp.transpose(theta_w, (0, 2, 1))
    w2 = jnp.transpose(phi_w - theta_w, (0, 2, 1))
    b = jnp.broadcast_to((theta_b + phi_b).reshape(L, 1, D), (L, 8, D))

    xpad = jnp.concatenate(
        [feats, jnp.zeros((NPAD - N, D), jnp.float32)], axis=0
    )
    neg = jnp.full((NP + 1, D), -jnp.inf, jnp.float32)

    u, v = _mm_first_call(xpad, w1[0], w2[0], b[0])
    for l in range(1, L):
        agg = _seg_max(u, bsrc, bdst, bcnt, neg)
        agg = agg.reshape(NPAD, D)
        u, v = _mm_fused_call(agg, v, w1[l], w2[l], b[l])
    agg = _seg_max(u, bsrc, bdst, bcnt, neg)
    y = _epilogue_call(agg.reshape(NPAD, D), v)
    return y[:N]


# speedup vs baseline: 1.3499x; 1.0216x over previous
import functools

import jax
import jax.numpy as jnp
from jax import lax
from jax.experimental import pallas as pl
from jax.experimental.pallas import tpu as pltpu
from jax.experimental.pallas import tpu_sc as plsc

N = 10000
E = 320000
D = 128
L = 4

NC, NS, LANES = 2, 16, 16
NW = NC * NS
NP = 313
NPAD = NW * NP
DUMP = NP

EC = 16000
F = 12800
S = F + EC + 784
SH = S - F
CAP = E + 2 * F
GCH = 256
MAXSORT = F + 16768
HB = 352

_sc_params = pltpu.CompilerParams(needs_layout_passes=False)
_mesh = plsc.VectorSubcoreMesh(core_axis_name="c", subcore_axis_name="s")


@functools.partial(
    pl.kernel,
    mesh=_mesh,
    compiler_params=_sc_params,
    out_type=(
        jax.ShapeDtypeStruct((NW * CAP,), jnp.int32),
        jax.ShapeDtypeStruct((NW * CAP,), jnp.int32),
        jax.ShapeDtypeStruct((NW * 128,), jnp.int32),
    ),
    scratch_types=[
        pltpu.VMEM((EC + LANES,), jnp.int32),
        pltpu.VMEM((EC + LANES,), jnp.int32),
        pltpu.VMEM((EC + LANES,), jnp.int32),
        pltpu.VMEM((EC + LANES,), jnp.int32),
        pltpu.VMEM((S,), jnp.int32),
        pltpu.VMEM((S,), jnp.int32),
        pltpu.SemaphoreType.DMA,
        pltpu.SemaphoreType.DMA,
        pltpu.SemaphoreType.DMA,
        pltpu.SemaphoreType.DMA,
        pltpu.VMEM((HB,), jnp.int32),
        pltpu.VMEM((HB,), jnp.int32),
        pltpu.VMEM((HB,), jnp.int32),
        pltpu.VMEM((HB,), jnp.int32),
        pltpu.VMEM((HB,), jnp.int32),
        pltpu.VMEM((HB,), jnp.int32),
        pltpu.VMEM((HB,), jnp.int32),
        pltpu.VMEM((HB,), jnp.int32),
    ],
)
def _bin_edges(src_hbm, dst_hbm, bsrc, bdst, bcnt, src_c, src_c1, dst_c, dst_c1,
               st_src, st_dst, sa0, sa1, sb0, sb1, h0, h1, h2, h3, o0, o1, o2, o3):
    wid = lax.axis_index("s") * NC + lax.axis_index("c")
    lo = wid * NP
    iota = jnp.arange(LANES, dtype=jnp.int32)
    lane0 = iota == 0
    z16 = jnp.zeros((LANES,), jnp.int32)

    sbuf = (src_c, src_c1)
    dbuf = (dst_c, dst_c1)
    asem = (sa0, sa1)
    bsem = (sb0, sb1)

    def scan_off(g):
        return pl.multiple_of(g * EC, 128)

    def issue_scan(g, b):
        pltpu.async_copy(src_hbm.at[pl.ds(scan_off(g), EC)], sbuf[b].at[pl.ds(0, EC)], asem[b])
        pltpu.async_copy(dst_hbm.at[pl.ds(scan_off(g), EC)], dbuf[b].at[pl.ds(0, EC)], bsem[b])

    issue_scan(0, 0)
    issue_scan(1, 1)

    def outer(g2, carry):
        for b in (0, 1):
            g = g2 * 2 + b
            cnt_st, flushed = carry
            pltpu.make_async_copy(
                src_hbm.at[pl.ds(scan_off(g), EC)], sbuf[b].at[pl.ds(0, EC)], asem[b]
            ).wait()
            pltpu.make_async_copy(
                dst_hbm.at[pl.ds(scan_off(g), EC)], dbuf[b].at[pl.ds(0, EC)], bsem[b]
            ).wait()

            def inner(i, cnt):
                sl = pl.ds(i * LANES, LANES)
                s = sbuf[b][sl]
                dl = dbuf[b][sl] - lo
                m = (dl >= 0) & (dl < NP)
                plsc.store_compressed(st_src.at[pl.ds(cnt, LANES)], s, mask=m)
                plsc.store_compressed(st_dst.at[pl.ds(cnt, LANES)], dl, mask=m)
                return cnt + plsc.all_reduce_population_count(m)[0]

            cnt_st = lax.fori_loop(0, EC // LANES, inner, cnt_st)

            @pl.when(g + 2 < E // EC)
            def _prefetch():
                issue_scan(g + 2, b)

            for _ in range(2):
                do = cnt_st >= F

                @pl.when(do)
                def _flush():
                    off = pl.multiple_of(wid * CAP + flushed, 128)
                    pltpu.sync_copy(st_src.at[pl.ds(0, F)], bsrc.at[pl.ds(off, F)])
                    pltpu.sync_copy(st_dst.at[pl.ds(0, F)], bdst.at[pl.ds(off, F)])

                    def shift(i, c):
                        sl_lo = pl.ds(i * LANES, LANES)
                        sl_hi = pl.ds(F + i * LANES, LANES)
                        st_src[sl_lo] = st_src[sl_hi]
                        st_dst[sl_lo] = st_dst[sl_hi]
                        return c

                    lax.fori_loop(0, SH // LANES, shift, 0)

                cnt_st = jnp.where(do, cnt_st - F, cnt_st)
                flushed = jnp.where(do, flushed + F, flushed)
            carry = (cnt_st, flushed)
        return carry

    cnt_st, flushed = lax.fori_loop(0, (E // EC) // 2, outer, (0, 0))

    pad_n = lax.rem(2 * GCH - lax.rem(cnt_st, 2 * GCH), 2 * GCH)
    pad_src = z16 + wid
    pad_dst = z16 + DUMP
    for j in range(2 * GCH // LANES):
        @pl.when(j * LANES < pad_n)
        def _pad():
            st_src[pl.ds(cnt_st + j * LANES, LANES)] = pad_src
            st_dst[pl.ds(cnt_st + j * LANES, LANES)] = pad_dst

    off = pl.multiple_of(wid * CAP + flushed, 128)
    pltpu.sync_copy(st_src.at[pl.ds(0, F)], bsrc.at[pl.ds(off, F)])
    pltpu.sync_copy(st_dst.at[pl.ds(0, F)], bdst.at[pl.ds(off, F)])

    total = flushed + cnt_st + pad_n
    sortable = total <= MAXSORT

    @pl.when(sortable)
    def _sort():
        hs = (h0, h1, h2, h3)
        os_ = (o0, o1, o2, o3)
        for hk in hs:
            for k in range(HB // LANES):
                hk[pl.ds(k * LANES, LANES)] = z16

        nct = lax.div(total + (EC - 1), EC)

        def hist_chunk(t, c):
            coff = pl.multiple_of(wid * CAP + t * EC, 128)
            pltpu.sync_copy(bdst.at[pl.ds(coff, EC)], dst_c.at[pl.ds(0, EC)])
            nb = jnp.minimum(EC, total - t * EC)

            def hist_edge(i, cc):
                for k in range(4):
                    d = dst_c[pl.ds(i * 4 + k, LANES)][0]
                    hcnt = hs[k][pl.ds(d, LANES)][0]
                    plsc.store_scatter(hs[k], [z16 + d], z16 + (hcnt + 1), mask=lane0)
                return cc

            lax.fori_loop(0, lax.div(nb, 4), hist_edge, 0)
            return c

        lax.fori_loop(0, nct, hist_chunk, 0)

        running = jnp.int32(0)
        for k in range(HB // LANES):
            sl = pl.ds(k * LANES, LANES)
            v0, v1, v2, v3 = h0[sl], h1[sl], h2[sl], h3[sl]
            hv = v0 + v1 + v2 + v3
            cs = plsc.cumsum(hv)
            base = cs - hv + running
            o0[sl] = base
            o1[sl] = base + v0
            o2[sl] = base + v0 + v1
            o3[sl] = base + v0 + v1 + v2
            running = running + cs[LANES - 1]

        def perm_chunk(t, c):
            coff = pl.multiple_of(wid * CAP + t * EC, 128)
            pltpu.sync_copy(bsrc.at[pl.ds(coff, EC)], src_c.at[pl.ds(0, EC)])
            pltpu.sync_copy(bdst.at[pl.ds(coff, EC)], dst_c.at[pl.ds(0, EC)])
            nb = jnp.minimum(EC, total - t * EC)

            def perm_edge(i, cc):
                for k in range(4):
                    s = src_c[pl.ds(i * 4 + k, LANES)][0]
                    d = dst_c[pl.ds(i * 4 + k, LANES)][0]
                    o = os_[k][pl.ds(d, LANES)][0]
                    plsc.store_scatter(os_[k], [z16 + d], z16 + (o + 1), mask=lane0)
                    plsc.store_scatter(st_src, [z16 + o], z16 + s, mask=lane0)
                    plsc.store_scatter(st_dst, [z16 + o], z16 + d, mask=lane0)
                return cc

            lax.fori_loop(0, lax.div(nb, 4), perm_edge, 0)
            return c

        lax.fori_loop(0, nct, perm_chunk, 0)

        base = pl.multiple_of(wid * CAP, 128)
        pltpu.sync_copy(st_src.at[pl.ds(0, F)], bsrc.at[pl.ds(base, F)])
        pltpu.sync_copy(st_dst.at[pl.ds(0, F)], bdst.at[pl.ds(base, F)])
        base2 = pl.multiple_of(wid * CAP + F, 128)
        pltpu.sync_copy(st_src.at[pl.ds(F, MAXSORT - F)], bsrc.at[pl.ds(base2, MAXSORT - F)])
        pltpu.sync_copy(st_dst.at[pl.ds(F, MAXSORT - F)], bdst.at[pl.ds(base2, MAXSORT - F)])

    flag = jnp.where(sortable, 1, 0)
    bcnt_v = jnp.where(iota == 0, z16 + total, jnp.where(iota == 1, z16 + flag, z16))
    src_c[pl.ds(0, LANES)] = bcnt_v
    pltpu.sync_copy(src_c.at[pl.ds(0, LANES)], bcnt.at[pl.ds(pl.multiple_of(wid * 128, 128), LANES)])


_NEG = float("-inf")


@functools.partial(
    pl.kernel,
    mesh=_mesh,
    compiler_params=_sc_params,
    out_type=jax.ShapeDtypeStruct((NW, NP, D), jnp.float32),
    scratch_types=[
        pltpu.VMEM((NW + LANES,), jnp.int32),
        pltpu.VMEM((GCH,), jnp.int32),
        pltpu.VMEM((GCH,), jnp.int32),
        pltpu.VMEM((GCH + LANES,), jnp.int32),
        pltpu.VMEM((GCH + LANES,), jnp.int32),
        pltpu.VMEM((GCH, D), jnp.float32),
        pltpu.VMEM((GCH, D), jnp.float32),
        pltpu.VMEM((NP + 1, D), jnp.float32),
        pltpu.SemaphoreType.DMA,
        pltpu.SemaphoreType.DMA,
        pltpu.SemaphoreType.DMA,
        pltpu.SemaphoreType.DMA,
        pltpu.SemaphoreType.DMA,
        pltpu.SemaphoreType.DMA,
    ],
)
def _seg_max(u_hbm, bsrc, bdst, bcnt, neg_hbm, agg, cnt_v, sidx0, sidx1,
             dloc0, dloc1, rows0, rows1, acc, s0, s1, d0, d1, r0, r1):
    wid = lax.axis_index("s") * NC + lax.axis_index("c")

    pltpu.sync_copy(bcnt.at[pl.ds(pl.multiple_of(wid * 128, 128), LANES)], cnt_v.at[pl.ds(0, LANES)])
    hdr = cnt_v[pl.ds(0, LANES)]
    cnt = hdr[0]
    flag = hdr[1]
    pltpu.sync_copy(neg_hbm, acc)
    nch = lax.div(cnt, GCH)

    sidx = (sidx0, sidx1)
    dloc = (dloc0, dloc1)
    rows = (rows0, rows1)
    ssem = (s0, s1)
    dsem = (d0, d1)
    rsem = (r0, r1)

    def idx_off(g):
        return pl.multiple_of(wid * CAP + g * GCH, 128)

    @pl.when(flag == 1)
    def _sorted():
        @pl.when(nch >= 1)
        def _pro0():
            hs = pltpu.async_copy(bsrc.at[pl.ds(idx_off(0), GCH)], sidx0, s0)
            pltpu.async_copy(bdst.at[pl.ds(idx_off(0), GCH)], dloc0.at[pl.ds(0, GCH)], d0)
            hs.wait()
            pltpu.async_copy(u_hbm.at[sidx0], rows0, r0)

        @pl.when(nch >= 2)
        def _pro1():
            pltpu.async_copy(bsrc.at[pl.ds(idx_off(1), GCH)], sidx1, s1)
            pltpu.async_copy(bdst.at[pl.ds(idx_off(1), GCH)], dloc1.at[pl.ds(0, GCH)], d1)

        def pair(g2, carry):
            for b in (0, 1):
                g = g2 * 2 + b
                nbuf = 1 - b
                pltpu.make_async_copy(u_hbm.at[sidx[b]], rows[b], rsem[b]).wait()

                @pl.when(g + 1 < nch)
                def _launch_next_gather():
                    pltpu.make_async_copy(
                        bsrc.at[pl.ds(idx_off(g + 1), GCH)], sidx[nbuf], ssem[nbuf]
                    ).wait()
                    pltpu.async_copy(u_hbm.at[sidx[nbuf]], rows[nbuf], rsem[nbuf])

                @pl.when(g + 2 < nch)
                def _prefetch_sidx():
                    pltpu.async_copy(
                        bsrc.at[pl.ds(idx_off(g + 2), GCH)], sidx[b], ssem[b]
                    )

                pltpu.make_async_copy(
                    bdst.at[pl.ds(idx_off(g), GCH)], dloc[b].at[pl.ds(0, GCH)], dsem[b]
                ).wait()

                def body(e, car):
                    prev = car[0]
                    ms = car[1:]
                    dst = dloc[b][pl.ds(e, LANES)][0]
                    same = dst == prev
                    new_ms = []
                    for r in range(D // LANES):
                        sl = pl.ds(r * LANES, LANES)
                        row_r = rows[b][e, sl]
                        new_ms.append(jnp.where(same, jnp.maximum(ms[r], row_r), row_r))
                    for r in range(D // LANES):
                        acc[dst, pl.ds(r * LANES, LANES)] = new_ms[r]
                    return (dst, *new_ms)

                carry = lax.fori_loop(0, GCH, body, carry)

                @pl.when(g + 2 < nch)
                def _prefetch_dloc():
                    pltpu.async_copy(
                        bdst.at[pl.ds(idx_off(g + 2), GCH)], dloc[b].at[pl.ds(0, GCH)], dsem[b]
                    )

            return carry

        init = (jnp.int32(-1),) + tuple(
            jnp.full((LANES,), _NEG, jnp.float32) for _ in range(D // LANES)
        )
        lax.fori_loop(0, lax.div(nch, 2), pair, init)

    @pl.when(flag == 0)
    def _rmw():
        def chunk(g, carry):
            pltpu.sync_copy(bsrc.at[pl.ds(idx_off(g), GCH)], sidx0)
            pltpu.sync_copy(bdst.at[pl.ds(idx_off(g), GCH)], dloc0.at[pl.ds(0, GCH)])
            pltpu.async_copy(u_hbm.at[sidx0], rows0, r0).wait()

            def body(e, c):
                dst = dloc0[pl.ds(e, LANES)][0]
                for r in range(D // LANES):
                    sl = pl.ds(r * LANES, LANES)
                    acc[dst, sl] = jnp.maximum(acc[dst, sl], rows0[e, sl])
                return c

            lax.fori_loop(0, GCH, body, 0)
            return carry

        lax.fori_loop(0, nch, chunk, 0)

    pltpu.sync_copy(acc.at[pl.ds(0, NP)], agg.at[wid])


def _mm_first(x_ref, w1_ref, w2_ref, b_ref, u_ref, v_ref):
    x = x_ref[...]
    u_ref[...] = jnp.dot(x, w1_ref[...], preferred_element_type=jnp.float32)
    v_ref[...] = (
        jnp.dot(x, w2_ref[...], preferred_element_type=jnp.float32) + b_ref[0:1]
    )


def _mm_fused(agg_ref, vin_ref, w1_ref, w2_ref, b_ref, u_ref, v_ref):
    x = jnp.maximum(agg_ref[...] + vin_ref[...], 0.0)
    u_ref[...] = jnp.dot(x, w1_ref[...], preferred_element_type=jnp.float32)
    v_ref[...] = (
        jnp.dot(x, w2_ref[...], preferred_element_type=jnp.float32) + b_ref[0:1]
    )


def _epilogue(agg_ref, vin_ref, y_ref):
    y_ref[...] = jnp.maximum(agg_ref[...] + vin_ref[...], 0.0)


_RB = NPAD // 4


def _row_spec():
    return pl.BlockSpec((_RB, D), lambda i: (i, 0))


def _full_spec(shape):
    return pl.BlockSpec(shape, lambda i: tuple(0 for _ in shape))


_mm_first_call = pl.pallas_call(
    _mm_first,
    grid=(4,),
    in_specs=[_row_spec(), _full_spec((D, D)), _full_spec((D, D)), _full_spec((8, D))],
    out_specs=[_row_spec(), _row_spec()],
    out_shape=(
        jax.ShapeDtypeStruct((NPAD, D), jnp.float32),
        jax.ShapeDtypeStruct((NPAD, D), jnp.float32),
    ),
)

_mm_fused_call = pl.pallas_call(
    _mm_fused,
    grid=(4,),
    in_specs=[
        _row_spec(),
        _row_spec(),
        _full_spec((D, D)),
        _full_spec((D, D)),
        _full_spec((8, D)),
    ],
    out_specs=[_row_spec(), _row_spec()],
    out_shape=(
        jax.ShapeDtypeStruct((NPAD, D), jnp.float32),
        jax.ShapeDtypeStruct((NPAD, D), jnp.float32),
    ),
)

_epilogue_call = pl.pallas_call(
    _epilogue,
    grid=(4,),
    in_specs=[_row_spec(), _row_spec()],
    out_specs=_row_spec(),
    out_shape=jax.ShapeDtypeStruct((NPAD, D), jnp.float32),
)


def kernel(feats, graph, theta_w, theta_b, phi_w, phi_b):
    src = graph[0].astype(jnp.int32)
    dst = graph[1].astype(jnp.int32)

    bsrc, bdst, bcnt = _bin_edges(src, dst)

    w1 = jnp.transpose(theta_w, (0, 2, 1))
    w2 = jnp.transpose(phi_w - theta_w, (0, 2, 1))
    b = jnp.broadcast_to((theta_b + phi_b).reshape(L, 1, D), (L, 8, D))

    xpad = jnp.concatenate(
        [feats, jnp.zeros((NPAD - N, D), jnp.float32)], axis=0
    )
    neg = jnp.full((NP + 1, D), -jnp.inf, jnp.float32)

    u, v = _mm_first_call(xpad, w1[0], w2[0], b[0])
    for l in range(1, L):
        agg = _seg_max(u, bsrc, bdst, bcnt, neg)
        agg = agg.reshape(NPAD, D)
        u, v = _mm_fused_call(agg, v, w1[l], w2[l], b[l])
    agg = _seg_max(u, bsrc, bdst, bcnt, neg)
    y = _epilogue_call(agg.reshape(NPAD, D), v)
    return y[:N]
